# baseline (speedup 1.0000x reference)
import functools

import jax
import jax.numpy as jnp
from jax import lax
from jax.experimental import pallas as pl
from jax.experimental.pallas import tpu as pltpu
from jax.experimental.pallas import tpu_sc as plsc

_NUM_EMB = 1001
_EMB_DIM = 64
_BATCH = 16384

_INFO = plsc.get_sparse_core_info()
_NC = _INFO.num_cores
_NS = _INFO.num_subcores
_NW = _NC * _NS
_L = _INFO.num_lanes

_NDIMG = 8
_NBATG = _NW // _NDIMG
_ROWS = _EMB_DIM // _NDIMG
_BCOLS = _BATCH // _NBATG
_NTILES = _BCOLS // 128
_VTILES = (_NUM_EMB + 127) // 128
_VLAST = _NUM_EMB - 128 * (_VTILES - 1)

_mesh = plsc.VectorSubcoreMesh(core_axis_name="c", subcore_axis_name="s")


@functools.partial(
    pl.kernel,
    mesh=_mesh,
    out_type=jax.ShapeDtypeStruct((_EMB_DIM, _BATCH), jnp.float32),
    scratch_types=[
        pltpu.VMEM((_VTILES * _ROWS, 128), jnp.float32),
        pltpu.VMEM((_ROWS, _VLAST), jnp.float32),
        pltpu.VMEM((_BCOLS,), jnp.int32),
        pltpu.VMEM((_ROWS, 128), jnp.float32),
        pltpu.VMEM((_ROWS, 128), jnp.float32),
        pltpu.SemaphoreType.DMA,
        pltpu.SemaphoreType.DMA,
    ],
    compiler_params=pltpu.CompilerParams(
        use_tc_tiling_on_sc=True, needs_layout_passes=False
    ),
)
def _sc_gather_t(idx_hbm, tt_hbm, out_hbm, tv, tlast, iv, tile_a, tile_b, sem_a, sem_b):
    wid = lax.axis_index("s") * _NC + lax.axis_index("c")
    g = wid % _NDIMG
    b = wid // _NDIMG
    rows = tt_hbm.at[pl.ds(g * _ROWS, _ROWS), :]
    stages = []
    for v in range(_VTILES - 1):
        stages.append(pltpu.async_copy(
            rows.at[:, pl.ds(v * 128, 128)], tv.at[pl.ds(v * _ROWS, _ROWS), :],
            sem_a,
        ))
    stages.append(pltpu.async_copy(
        rows.at[:, pl.ds((_VTILES - 1) * 128, _VLAST)], tlast, sem_a,
    ))
    pltpu.sync_copy(idx_hbm.at[pl.ds(b * _BCOLS, _BCOLS)], iv)
    for s in stages:
        s.wait()
    tail_offs = [c * _L for c in range(_VLAST // _L)] + [_VLAST - _L]
    for d in range(_ROWS):
        for o in tail_offs:
            tv[(_VTILES - 1) * _ROWS + d, pl.ds(o, _L)] = tlast[d, pl.ds(o, _L)]

    def build(tile, t):
        for c in range(128 // _L):
            ivec = iv[pl.ds(t * 128 + c * _L, _L)]
            hi = (ivec >> 7) << 3
            lo = ivec & 127
            rows_i = [hi] + [hi + d for d in range(1, _ROWS)]
            vals = [plsc.load_gather(tv, [r, lo]) for r in rows_i]
            for d in range(_ROWS):
                tile[d, pl.ds(c * _L, _L)] = vals[d]

    def out_slice(t):
        return out_hbm.at[pl.ds(g * _ROWS, _ROWS), pl.ds(b * _BCOLS + t * 128, 128)]

    build(tile_a, 0)

    def body(i, carry):
        t0 = 2 * i
        wa = pltpu.async_copy(tile_a, out_slice(t0), sem_a)
        build(tile_b, t0 + 1)
        wa.wait()
        wb = pltpu.async_copy(tile_b, out_slice(t0 + 1), sem_b)

        @pl.when(i < _NTILES // 2 - 1)
        def _():
            build(tile_a, t0 + 2)

        wb.wait()
        return carry

    lax.fori_loop(0, _NTILES // 2, body, jnp.int32(0))


def kernel(user_ids, table):
    out_t = _sc_gather_t(user_ids, table.T)
    return jnp.transpose(out_t)


# speedup vs baseline: 1.0668x; 1.0668x over previous
import functools

import jax
import jax.numpy as jnp
from jax import lax
from jax.experimental import pallas as pl
from jax.experimental.pallas import tpu as pltpu
from jax.experimental.pallas import tpu_sc as plsc

_NUM_EMB = 1001
_EMB_DIM = 64
_BATCH = 16384

_INFO = plsc.get_sparse_core_info()
_NC = _INFO.num_cores
_NS = _INFO.num_subcores
_NW = _NC * _NS
_L = _INFO.num_lanes

_NDIMG = 8
_NBATG = _NW // _NDIMG
_ROWS = _EMB_DIM // _NDIMG
_BCOLS = _BATCH // _NBATG
_TPAD = 1008
_NTILES = _BCOLS // 128

_mesh = plsc.VectorSubcoreMesh(core_axis_name="c", subcore_axis_name="s")


@functools.partial(
    pl.kernel,
    mesh=_mesh,
    out_type=jax.ShapeDtypeStruct((_EMB_DIM, _BATCH), jnp.float32),
    scratch_types=[
        pltpu.VMEM((_ROWS * _TPAD,), jnp.float32),
        pltpu.VMEM((_BCOLS,), jnp.int32),
        pltpu.VMEM((_ROWS, 128), jnp.float32),
        pltpu.VMEM((_ROWS, 128), jnp.float32),
        pltpu.SemaphoreType.DMA,
        pltpu.SemaphoreType.DMA,
    ],
    compiler_params=pltpu.CompilerParams(
        use_tc_tiling_on_sc=True, needs_layout_passes=False
    ),
)
def _sc_gather_t(idx_hbm, tflat_hbm, out_hbm, tv, iv, tile_a, tile_b, sem_a, sem_b):
    wid = lax.axis_index("s") * _NC + lax.axis_index("c")
    g = wid % _NDIMG
    b = wid // _NDIMG
    st = pltpu.async_copy(
        tflat_hbm.at[pl.ds(g * _ROWS * _TPAD, _ROWS * _TPAD)], tv, sem_a
    )
    si = pltpu.async_copy(idx_hbm.at[pl.ds(b * _BCOLS, _BCOLS)], iv, sem_b)
    st.wait()
    si.wait()

    def build(tile, t):
        for c in range(128 // _L):
            ivec = iv[pl.ds(t * 128 + c * _L, _L)]
            addrs = [ivec] + [ivec + d * _TPAD for d in range(1, _ROWS)]
            vals = [plsc.load_gather(tv, [a]) for a in addrs]
            for d in range(_ROWS):
                tile[d, pl.ds(c * _L, _L)] = vals[d]

    def out_slice(t):
        return out_hbm.at[pl.ds(g * _ROWS, _ROWS), pl.ds(b * _BCOLS + t * 128, 128)]

    build(tile_a, 0)

    def body(i, carry):
        t0 = 2 * i
        wa = pltpu.async_copy(tile_a, out_slice(t0), sem_a)
        build(tile_b, t0 + 1)
        wa.wait()
        wb = pltpu.async_copy(tile_b, out_slice(t0 + 1), sem_b)

        @pl.when(i < _NTILES // 2 - 1)
        def _():
            build(tile_a, t0 + 2)

        wb.wait()
        return carry

    lax.fori_loop(0, _NTILES // 2, body, jnp.int32(0))


def kernel(user_ids, table):
    tflat = jnp.pad(table.T, ((0, 0), (0, _TPAD - _NUM_EMB))).reshape(-1)
    out_t = _sc_gather_t(user_ids, tflat)
    return jnp.transpose(out_t)


# speedup vs baseline: 1.1189x; 1.0488x over previous
import functools

import jax
import jax.numpy as jnp
from jax import lax
from jax.experimental import pallas as pl
from jax.experimental.pallas import tpu as pltpu
from jax.experimental.pallas import tpu_sc as plsc

_NUM_EMB = 1001
_EMB_DIM = 64
_BATCH = 16384

_INFO = plsc.get_sparse_core_info()
_NC = _INFO.num_cores
_NS = _INFO.num_subcores
_NW = _NC * _NS
_L = _INFO.num_lanes

_NDIMG = 8
_NBATG = _NW // _NDIMG
_ROWS = _EMB_DIM // _NDIMG
_BCOLS = _BATCH // _NBATG
_TPAD = 1008
_NTILES = _BCOLS // 128

_mesh = plsc.VectorSubcoreMesh(core_axis_name="c", subcore_axis_name="s")


@functools.partial(
    pl.kernel,
    mesh=_mesh,
    out_type=jax.ShapeDtypeStruct((_EMB_DIM, _BATCH), jnp.float32),
    scratch_types=[
        pltpu.VMEM((_ROWS * _TPAD,), jnp.float32),
        pltpu.VMEM((_BCOLS,), jnp.int32),
        pltpu.VMEM((_ROWS, 128), jnp.float32),
        pltpu.VMEM((_ROWS, 128), jnp.float32),
        pltpu.SemaphoreType.DMA,
        pltpu.SemaphoreType.DMA,
    ],
    compiler_params=pltpu.CompilerParams(
        use_tc_tiling_on_sc=True, needs_layout_passes=False
    ),
)
def _sc_gather_t(idx_hbm, tflat_hbm, out_hbm, tv, iv, tile_a, tile_b, sem_a, sem_b):
    wid = lax.axis_index("s") * _NC + lax.axis_index("c")
    g = wid % _NDIMG
    b = wid // _NDIMG
    st = pltpu.async_copy(
        tflat_hbm.at[pl.ds(g * _ROWS * _TPAD, _ROWS * _TPAD)], tv, sem_a
    )
    si = pltpu.async_copy(idx_hbm.at[pl.ds(b * _BCOLS, _BCOLS)], iv, sem_b)
    st.wait()
    si.wait()

    def build(tile, t):
        @plsc.parallel_loop(0, 128 // _L, unroll=128 // _L)
        def _(c):
            ivec = iv[pl.ds(t * 128 + c * _L, _L)]
            addrs = [ivec] + [ivec + d * _TPAD for d in range(1, _ROWS)]
            vals = [plsc.load_gather(tv, [a]) for a in addrs]
            for d in range(_ROWS):
                tile[d, pl.ds(c * _L, _L)] = vals[d]

    def out_slice(t):
        return out_hbm.at[pl.ds(g * _ROWS, _ROWS), pl.ds(b * _BCOLS + t * 128, 128)]

    build(tile_a, 0)

    def body(i, carry):
        t0 = 2 * i
        wa = pltpu.async_copy(tile_a, out_slice(t0), sem_a)
        build(tile_b, t0 + 1)
        wa.wait()
        wb = pltpu.async_copy(tile_b, out_slice(t0 + 1), sem_b)

        @pl.when(i < _NTILES // 2 - 1)
        def _():
            build(tile_a, t0 + 2)

        wb.wait()
        return carry

    lax.fori_loop(0, _NTILES // 2, body, jnp.int32(0))


def kernel(user_ids, table):
    tflat = jnp.pad(table.T, ((0, 0), (0, _TPAD - _NUM_EMB))).reshape(-1)
    out_t = _sc_gather_t(user_ids, tflat)
    return jnp.transpose(out_t)


# speedup vs baseline: 1.1303x; 1.0102x over previous
import functools

import jax
import jax.numpy as jnp
from jax import lax
from jax.experimental import pallas as pl
from jax.experimental.pallas import tpu as pltpu
from jax.experimental.pallas import tpu_sc as plsc

_NUM_EMB = 1001
_EMB_DIM = 64
_BATCH = 16384

_INFO = plsc.get_sparse_core_info()
_NC = _INFO.num_cores
_NS = _INFO.num_subcores
_NW = _NC * _NS
_L = _INFO.num_lanes

_NDIMG = 8
_NBATG = _NW // _NDIMG
_ROWS = _EMB_DIM // _NDIMG
_BCOLS = _BATCH // _NBATG
_TPAD = 1008
_NTILES = _BCOLS // 128

_mesh = plsc.VectorSubcoreMesh(core_axis_name="c", subcore_axis_name="s")


@functools.partial(
    pl.kernel,
    mesh=_mesh,
    out_type=jax.ShapeDtypeStruct((_EMB_DIM, _BATCH), jnp.float32),
    scratch_types=[
        pltpu.VMEM((_ROWS * _TPAD,), jnp.float32),
        pltpu.VMEM((_BCOLS,), jnp.int32),
        pltpu.VMEM((_ROWS, 128), jnp.float32),
        pltpu.VMEM((_ROWS, 128), jnp.float32),
        pltpu.VMEM((_ROWS, 128), jnp.float32),
        pltpu.VMEM((_ROWS, 128), jnp.float32),
        pltpu.SemaphoreType.DMA,
        pltpu.SemaphoreType.DMA,
        pltpu.SemaphoreType.DMA,
        pltpu.SemaphoreType.DMA,
    ],
    compiler_params=pltpu.CompilerParams(
        use_tc_tiling_on_sc=True, needs_layout_passes=False
    ),
)
def _sc_gather_t(
    idx_hbm, tflat_hbm, out_hbm, tv, iv,
    tile_a, tile_b, tile_c, tile_d, sem_a, sem_b, sem_c, sem_d,
):
    wid = lax.axis_index("s") * _NC + lax.axis_index("c")
    g = wid % _NDIMG
    b = wid // _NDIMG
    st = pltpu.async_copy(
        tflat_hbm.at[pl.ds(g * _ROWS * _TPAD, _ROWS * _TPAD)], tv, sem_a
    )
    si = pltpu.async_copy(idx_hbm.at[pl.ds(b * _BCOLS, _BCOLS)], iv, sem_b)
    st.wait()
    si.wait()

    def build(tile, t):
        @plsc.parallel_loop(0, 128 // _L, unroll=128 // _L)
        def _(c):
            ivec = iv[pl.ds(t * 128 + c * _L, _L)]
            addrs = [ivec] + [ivec + d * _TPAD for d in range(1, _ROWS)]
            vals = [plsc.load_gather(tv, [a]) for a in addrs]
            for d in range(_ROWS):
                tile[d, pl.ds(c * _L, _L)] = vals[d]

    def out_slice(t):
        return out_hbm.at[pl.ds(g * _ROWS, _ROWS), pl.ds(b * _BCOLS + t * 128, 128)]

    build(tile_a, 0)
    build(tile_b, 1)

    def body(i, carry):
        t0 = 4 * i
        wa = pltpu.async_copy(tile_a, out_slice(t0), sem_a)
        wb = pltpu.async_copy(tile_b, out_slice(t0 + 1), sem_b)
        build(tile_c, t0 + 2)
        build(tile_d, t0 + 3)
        wa.wait()
        wb.wait()
        wc = pltpu.async_copy(tile_c, out_slice(t0 + 2), sem_c)
        wd = pltpu.async_copy(tile_d, out_slice(t0 + 3), sem_d)

        @pl.when(i < _NTILES // 4 - 1)
        def _():
            build(tile_a, t0 + 4)
            build(tile_b, t0 + 5)

        wc.wait()
        wd.wait()
        return carry

    lax.fori_loop(0, _NTILES // 4, body, jnp.int32(0))


def kernel(user_ids, table):
    tflat = jnp.pad(table.T, ((0, 0), (0, _TPAD - _NUM_EMB))).reshape(-1)
    out_t = _sc_gather_t(user_ids, tflat)
    return jnp.transpose(out_t)
